# baseline (speedup 1.0000x reference)
import functools

import jax
import jax.numpy as jnp
from jax import lax
from jax.experimental import pallas as pl
from jax.experimental.pallas import tpu as pltpu
from jax.experimental.pallas import tpu_sc as plsc

B, N, D = 4096, 19, 512
R = 4
STARTS = (0, 7, 12, 17)
ENDS = (7, 12, 17, 19)
SCALES = (1.0 / 7.0, 1.0 / 5.0, 1.0 / 5.0, 1.0 / 2.0)

NC, NS = 2, 16
NW = NC * NS
BPW = B // NW
CB = 4
NCHUNK = BPW // CB
LANES = 16
DCHUNKS = D // LANES


def _compute_chunk(ibuf, obuf):
    def bbody(b, _):
        for dc in range(DCHUNKS):
            off = dc * LANES
            v = [ibuf[b, c, pl.ds(off, LANES)] for c in range(N)]
            for r in range(R):
                acc = v[STARTS[r]]
                for c in range(STARTS[r] + 1, ENDS[r]):
                    acc = acc + v[c]
                obuf[b, r, pl.ds(off, LANES)] = acc * jnp.float32(SCALES[r])
        return _
    lax.fori_loop(0, CB, bbody, None)


def _pool_body(x_hbm, out_hbm, in0, in1, ob0, ob1, isem0, isem1, osem0, osem1):
    wid = lax.axis_index("s") * NC + lax.axis_index("c")
    base = wid * BPW

    def start_in(g, buf, sem):
        pltpu.async_copy(x_hbm.at[pl.ds(base + g * CB, CB)], buf, sem)

    def wait_in(g, buf, sem):
        pltpu.make_async_copy(x_hbm.at[pl.ds(base + g * CB, CB)], buf, sem).wait()

    def start_out(g, buf, sem):
        pltpu.async_copy(buf, out_hbm.at[pl.ds(base + g * CB, CB)], sem)

    def wait_out(g, buf, sem):
        pltpu.make_async_copy(buf, out_hbm.at[pl.ds(base + g * CB, CB)], sem).wait()

    start_in(0, in0, isem0)

    def gbody(h, _):
        g = h * 2
        wait_in(g, in0, isem0)

        @pl.when(g + 1 < NCHUNK)
        def _():
            start_in(g + 1, in1, isem1)

        @pl.when(g >= 2)
        def _():
            wait_out(g - 2, ob0, osem0)

        _compute_chunk(in0, ob0)
        start_out(g, ob0, osem0)

        wait_in(g + 1, in1, isem1)

        @pl.when(g + 2 < NCHUNK)
        def _():
            start_in(g + 2, in0, isem0)

        @pl.when(g >= 2)
        def _():
            wait_out(g - 1, ob1, osem1)

        _compute_chunk(in1, ob1)
        start_out(g + 1, ob1, osem1)
        return _

    lax.fori_loop(0, NCHUNK // 2, gbody, None)
    wait_out(NCHUNK - 2, ob0, osem0)
    wait_out(NCHUNK - 1, ob1, osem1)


_pool = functools.partial(
    pl.kernel,
    out_type=jax.ShapeDtypeStruct((B, R, D), jnp.float32),
    mesh=plsc.VectorSubcoreMesh(core_axis_name="c", subcore_axis_name="s"),
    scratch_types=[
        pltpu.VMEM((CB, N, D), jnp.float32),
        pltpu.VMEM((CB, N, D), jnp.float32),
        pltpu.VMEM((CB, R, D), jnp.float32),
        pltpu.VMEM((CB, R, D), jnp.float32),
        pltpu.SemaphoreType.DMA,
        pltpu.SemaphoreType.DMA,
        pltpu.SemaphoreType.DMA,
        pltpu.SemaphoreType.DMA,
    ],
)(_pool_body)


@jax.jit
def kernel(node_embeddings):
    return _pool(node_embeddings)


# speedup vs baseline: 3.0487x; 3.0487x over previous
import functools

import jax
import jax.numpy as jnp
from jax import lax
from jax.experimental import pallas as pl
from jax.experimental.pallas import tpu as pltpu
from jax.experimental.pallas import tpu_sc as plsc

B, N, D = 4096, 19, 512
R = 4
STARTS = (0, 7, 12, 17)
ENDS = (7, 12, 17, 19)
SCALES = (1.0 / 7.0, 1.0 / 5.0, 1.0 / 5.0, 1.0 / 2.0)

NC, NS = 2, 16
NW = NC * NS
BPW = B // NW
CBB = 8
DHALF = 256
NUNIT = (BPW // CBB) * (D // DHALF)
LANES = 16
DCHUNKS = DHALF // LANES


def _compute_unit(ibuf, obuf):
    def bbody(b, _):
        for dc in range(DCHUNKS):
            off = dc * LANES
            v = [ibuf[c, b, pl.ds(off, LANES)] for c in range(N)]
            for r in range(R):
                acc = v[STARTS[r]]
                for c in range(STARTS[r] + 1, ENDS[r]):
                    acc = acc + v[c]
                obuf[b, r, pl.ds(off, LANES)] = acc * jnp.float32(SCALES[r])
        return _
    lax.fori_loop(0, CBB, bbody, None)


def _pool_body(xt_hbm, out_hbm, in0, in1, ob0, ob1, isem0, isem1, osem0, osem1):
    wid = lax.axis_index("s") * NC + lax.axis_index("c")
    base = wid * BPW

    def in_slice(u):
        b0 = base + (u // 2) * CBB
        d0 = (u % 2) * DHALF
        return xt_hbm.at[:, pl.ds(b0, CBB), pl.ds(d0, DHALF)]

    def out_slice(u):
        b0 = base + (u // 2) * CBB
        d0 = (u % 2) * DHALF
        return out_hbm.at[pl.ds(b0, CBB), :, pl.ds(d0, DHALF)]

    def start_in(u, buf, sem):
        pltpu.async_copy(in_slice(u), buf, sem)

    def wait_in(u, buf, sem):
        pltpu.make_async_copy(in_slice(u), buf, sem).wait()

    def start_out(u, buf, sem):
        pltpu.async_copy(buf, out_slice(u), sem)

    def wait_out(u, buf, sem):
        pltpu.make_async_copy(buf, out_slice(u), sem).wait()

    start_in(0, in0, isem0)

    def ubody(h, _):
        u = h * 2
        wait_in(u, in0, isem0)

        @pl.when(u + 1 < NUNIT)
        def _():
            start_in(u + 1, in1, isem1)

        @pl.when(u >= 2)
        def _():
            wait_out(u - 2, ob0, osem0)

        _compute_unit(in0, ob0)
        start_out(u, ob0, osem0)

        wait_in(u + 1, in1, isem1)

        @pl.when(u + 2 < NUNIT)
        def _():
            start_in(u + 2, in0, isem0)

        @pl.when(u >= 2)
        def _():
            wait_out(u - 1, ob1, osem1)

        _compute_unit(in1, ob1)
        start_out(u + 1, ob1, osem1)
        return _

    lax.fori_loop(0, NUNIT // 2, ubody, None)
    wait_out(NUNIT - 2, ob0, osem0)
    wait_out(NUNIT - 1, ob1, osem1)


_pool = functools.partial(
    pl.kernel,
    out_type=jax.ShapeDtypeStruct((B, R, D), jnp.float32),
    mesh=plsc.VectorSubcoreMesh(core_axis_name="c", subcore_axis_name="s"),
    scratch_types=[
        pltpu.VMEM((N, CBB, DHALF), jnp.float32),
        pltpu.VMEM((N, CBB, DHALF), jnp.float32),
        pltpu.VMEM((CBB, R, DHALF), jnp.float32),
        pltpu.VMEM((CBB, R, DHALF), jnp.float32),
        pltpu.SemaphoreType.DMA,
        pltpu.SemaphoreType.DMA,
        pltpu.SemaphoreType.DMA,
        pltpu.SemaphoreType.DMA,
    ],
)(_pool_body)


@jax.jit
def kernel(node_embeddings):
    x_t = jnp.transpose(node_embeddings, (1, 0, 2))
    return _pool(x_t)
